# Initial kernel scaffold; baseline (speedup 1.0000x reference)
import jax
import jax.numpy as jnp
from jax.experimental import pallas as pl


def kernel(hidden_states, gate_w, w_gate, w_up, w_down):
    raise NotImplementedError("write your pallas kernel here")


# speedup vs baseline: 1.7045x; 1.7045x over previous
import jax
import jax.numpy as jnp
from jax.experimental import pallas as pl
from jax.experimental.pallas import tpu as pltpu

_T, _D, _E, _K, _F = 2048, 2048, 8, 2, 768
_BF = 256


def _router_body(x_ref, gate_w_ref, w_ref):
    logits = jnp.dot(x_ref[...], gate_w_ref[...].T, preferred_element_type=jnp.float32)
    m = jnp.max(logits, axis=1, keepdims=True)
    p = jnp.exp(logits - m)
    p = p / jnp.sum(p, axis=1, keepdims=True)
    iota = jax.lax.broadcasted_iota(jnp.int32, (_T, _E), 1)
    m1 = jnp.max(p, axis=1, keepdims=True)
    i1 = jnp.min(jnp.where(p == m1, iota, _E), axis=1, keepdims=True)
    mask1 = iota == i1
    p2 = jnp.where(mask1, -1.0, p)
    m2 = jnp.max(p2, axis=1, keepdims=True)
    i2 = jnp.min(jnp.where(p2 == m2, iota, _E), axis=1, keepdims=True)
    sel = mask1 | (iota == i2)
    w_ref[...] = jnp.where(sel, p, 0.0) / (m1 + m2)


def _expert_body(x_ref, w_ref, wg_ref, wu_ref, wd_ref, out_ref):
    e = pl.program_id(0)
    fb = pl.program_id(1)
    first = (e == 0) & (fb == 0)

    x = x_ref[...]
    g = jnp.dot(x, wg_ref[0].astype(jnp.bfloat16).T, preferred_element_type=jnp.float32)
    u = jnp.dot(x, wu_ref[0].astype(jnp.bfloat16).T, preferred_element_type=jnp.float32)
    iota = jax.lax.broadcasted_iota(jnp.int32, (_T, _E), 1)
    wcol = jnp.sum(jnp.where(iota == e, w_ref[...], 0.0), axis=1, keepdims=True)
    a = (jax.nn.silu(g) * u * wcol).astype(jnp.bfloat16)
    part = jnp.dot(a, wd_ref[0].astype(jnp.bfloat16).T, preferred_element_type=jnp.float32)

    @pl.when(first)
    def _init():
        out_ref[...] = part

    @pl.when(~first)
    def _acc():
        out_ref[...] += part


def kernel(hidden_states, gate_w, w_gate, w_up, w_down):
    x = hidden_states.reshape(-1, _D)
    router_w = pl.pallas_call(
        _router_body,
        in_specs=[
            pl.BlockSpec((_T, _D), lambda: (0, 0)),
            pl.BlockSpec((_E, _D), lambda: (0, 0)),
        ],
        out_specs=pl.BlockSpec((_T, _E), lambda: (0, 0)),
        out_shape=jax.ShapeDtypeStruct((_T, _E), jnp.float32),
    )(x, gate_w)

    x16 = x.astype(jnp.bfloat16)
    out = pl.pallas_call(
        _expert_body,
        grid=(_E, _F // _BF),
        in_specs=[
            pl.BlockSpec((_T, _D), lambda e, fb: (0, 0)),
            pl.BlockSpec((_T, _E), lambda e, fb: (0, 0)),
            pl.BlockSpec((1, _BF, _D), lambda e, fb: (e, fb, 0)),
            pl.BlockSpec((1, _BF, _D), lambda e, fb: (e, fb, 0)),
            pl.BlockSpec((1, _D, _BF), lambda e, fb: (e, 0, fb)),
        ],
        out_specs=pl.BlockSpec((_T, _D), lambda e, fb: (0, 0)),
        out_shape=jax.ShapeDtypeStruct((_T, _D), jnp.float32),
        compiler_params=pltpu.CompilerParams(
            dimension_semantics=("arbitrary", "arbitrary"),
        ),
    )(x16, router_w, w_gate, w_up, w_down)
    return out.reshape(hidden_states.shape)
